# Initial kernel scaffold; baseline (speedup 1.0000x reference)
import jax
import jax.numpy as jnp
from jax.experimental import pallas as pl


def kernel(layers_num, user_emb, entity_emb, inter_edge, inter_edge_w, edge_index, edge_type, relation_emb, W_Q):
    raise NotImplementedError("write your pallas kernel here")


# speedup vs baseline: 2.8892x; 2.8892x over previous
import functools
import math

import jax
import jax.numpy as jnp
from jax import lax
from jax.experimental import pallas as pl
from jax.experimental.pallas import tpu as pltpu
from jax.experimental.pallas import tpu_sc as plsc

N_ENT = 10000
N_USR = 10000
DIMS = 128
N_REL = 16
N_HEADS = 2
D_K = DIMS // N_HEADS
LAYERS = 2

NC = 2
NS = 16
NW = NC * NS
CH = 128
CB = 2048

_MESH = dict(core_axis_name="c", subcore_axis_name="s", num_cores=NC,
             num_subcores=NS)


@functools.partial(jax.jit, static_argnames=("k_per_w", "d"))
def _sc_gather(table, idx2d, *, k_per_w, d):
    b = NW * k_per_w * CH
    mesh = plsc.VectorSubcoreMesh(**_MESH)

    @functools.partial(
        pl.kernel,
        out_type=jax.ShapeDtypeStruct((b, d), jnp.float32),
        mesh=mesh,
        scratch_types=[
            pltpu.VMEM((k_per_w, CH), jnp.int32),
            pltpu.VMEM((CH, d), jnp.float32),
            pltpu.SemaphoreType.DMA,
        ],
        compiler_params=pltpu.CompilerParams(needs_layout_passes=False),
    )
    def k(table_hbm, idx_hbm, out_hbm, idx_v, rows_v, sem):
        wid = lax.axis_index("s") * NC + lax.axis_index("c")
        pltpu.sync_copy(idx_hbm.at[pl.ds(wid * k_per_w, k_per_w)], idx_v)

        def step(j, carry):
            pltpu.async_copy(table_hbm.at[idx_v.at[j]], rows_v, sem).wait()
            pltpu.sync_copy(rows_v,
                            out_hbm.at[pl.ds((wid * k_per_w + j) * CH, CH)])
            return carry

        lax.fori_loop(0, k_per_w, step, 0)

    return k(table, idx2d)


@functools.partial(jax.jit, static_argnames=("k_per_w", "n_pad"))
def _sc_scatter_add(values, idx2d, zeros, *, k_per_w, n_pad):
    mesh = plsc.VectorSubcoreMesh(**_MESH)
    rpt = n_pad // NS

    @functools.partial(
        pl.kernel,
        out_type=jax.ShapeDtypeStruct((NC * n_pad, DIMS), jnp.float32),
        mesh=mesh,
        scratch_types=[
            pltpu.VMEM((k_per_w, CH), jnp.int32),
            pltpu.VMEM((CH, DIMS), jnp.float32),
            pltpu.VMEM_SHARED((n_pad, DIMS), jnp.float32),
            pltpu.SemaphoreType.DMA,
        ],
        compiler_params=pltpu.CompilerParams(needs_layout_passes=False),
    )
    def k(val_hbm, idx_hbm, zero_hbm, out_hbm, idx_v, val_v, acc_sh, sem):
        c = lax.axis_index("c")
        s = lax.axis_index("s")
        wid = s * NC + c
        pltpu.sync_copy(zero_hbm.at[pl.ds(s * rpt, rpt)],
                        acc_sh.at[pl.ds(s * rpt, rpt)])
        pltpu.sync_copy(idx_hbm.at[pl.ds(wid * k_per_w, k_per_w)], idx_v)
        plsc.subcore_barrier()

        def step(j, carry):
            pltpu.sync_copy(val_hbm.at[pl.ds((wid * k_per_w + j) * CH, CH)],
                            val_v)
            pltpu.sync_copy(val_v, acc_sh.at[idx_v.at[j]], add=True)
            return carry

        lax.fori_loop(0, k_per_w, step, 0)
        plsc.subcore_barrier()
        pltpu.sync_copy(acc_sh.at[pl.ds(s * rpt, rpt)],
                        out_hbm.at[pl.ds(c * n_pad + s * rpt, rpt)])

    return k(values, idx2d, zeros)


@functools.partial(jax.jit, static_argnames=("k_per_w", "n_pad"))
def _sc_ssum_scatter(ex2, idx2d, zeros, *, k_per_w, n_pad):
    mesh = plsc.VectorSubcoreMesh(**_MESH)
    rpt = n_pad // NS
    nex = k_per_w * CH * 2

    @functools.partial(
        pl.kernel,
        out_type=jax.ShapeDtypeStruct((NC * n_pad, DIMS), jnp.float32),
        mesh=mesh,
        scratch_types=[
            pltpu.VMEM((k_per_w, CH), jnp.int32),
            pltpu.VMEM((k_per_w, 2 * CH), jnp.float32),
            pltpu.VMEM((CH, DIMS), jnp.float32),
            pltpu.VMEM_SHARED((n_pad, DIMS), jnp.float32),
            pltpu.SemaphoreType.DMA,
        ],
        compiler_params=pltpu.CompilerParams(needs_layout_passes=False),
    )
    def k(ex_hbm, idx_hbm, zero_hbm, out_hbm, idx_v, ex_v, val_v, acc_sh,
          sem):
        c = lax.axis_index("c")
        s = lax.axis_index("s")
        wid = s * NC + c
        pltpu.sync_copy(zero_hbm.at[pl.ds(s * rpt, rpt)],
                        acc_sh.at[pl.ds(s * rpt, rpt)])
        pltpu.sync_copy(idx_hbm.at[pl.ds(wid * k_per_w, k_per_w)], idx_v)
        pltpu.sync_copy(ex_hbm.at[pl.ds(wid * k_per_w, k_per_w)], ex_v)

        def zrow(i, carry):
            for t in range(DIMS // 16):
                val_v[i, pl.ds(t * 16, 16)] = jnp.zeros((16,), jnp.float32)
            return carry

        lax.fori_loop(0, CH, zrow, 0)
        plsc.subcore_barrier()
        lane = lax.iota(jnp.int32, 16)
        c0 = (lane == 0).astype(jnp.float32)
        c1 = (lane == 1).astype(jnp.float32)

        def step(j, carry):
            def fill_grp(g, cc):
                for half in range(2):
                    p = ex_v[j, pl.ds(32 * g + 16 * half, 16)]
                    for l in range(8):
                        e = 16 * g + 8 * half + l
                        v0 = jnp.sum(p * (lane == 2 * l).astype(jnp.float32))
                        v1 = jnp.sum(p * (lane == 2 * l + 1).astype(
                            jnp.float32))
                        val_v[e, pl.ds(0, 16)] = (jnp.full((16,), v0) * c0 +
                                                  jnp.full((16,), v1) * c1)
                return cc

            lax.fori_loop(0, CH // 16, fill_grp, 0)
            pltpu.sync_copy(val_v, acc_sh.at[idx_v.at[j]], add=True)
            return carry

        lax.fori_loop(0, k_per_w, step, 0)
        plsc.subcore_barrier()
        pltpu.sync_copy(acc_sh.at[pl.ds(s * rpt, rpt)],
                        out_hbm.at[pl.ds(c * n_pad + s * rpt, rpt)])

    return k(ex2.reshape(NW * k_per_w, 2 * CH), idx2d, zeros)


@functools.partial(jax.jit, static_argnames=("k_per_w", "n_pad"))
def _sc_gather_scale_scatter(table, src2d, w2d, dst2d, zeros, *,
                             k_per_w, n_pad):
    mesh = plsc.VectorSubcoreMesh(**_MESH)
    rpt = n_pad // NS

    @functools.partial(
        pl.kernel,
        out_type=jax.ShapeDtypeStruct((NC * n_pad, DIMS), jnp.float32),
        mesh=mesh,
        scratch_types=[
            pltpu.VMEM((k_per_w, CH), jnp.int32),
            pltpu.VMEM((k_per_w, CH), jnp.int32),
            pltpu.VMEM((k_per_w, CH), jnp.float32),
            pltpu.VMEM((CH, DIMS), jnp.float32),
            pltpu.VMEM_SHARED((n_pad, DIMS), jnp.float32),
            pltpu.SemaphoreType.DMA,
        ],
        compiler_params=pltpu.CompilerParams(needs_layout_passes=False),
    )
    def k(table_hbm, src_hbm, w_hbm, dst_hbm, zero_hbm, out_hbm,
          src_v, dst_v, w_v, val_v, acc_sh, sem):
        c = lax.axis_index("c")
        s = lax.axis_index("s")
        wid = s * NC + c
        pltpu.sync_copy(zero_hbm.at[pl.ds(s * rpt, rpt)],
                        acc_sh.at[pl.ds(s * rpt, rpt)])
        pltpu.sync_copy(src_hbm.at[pl.ds(wid * k_per_w, k_per_w)], src_v)
        pltpu.sync_copy(dst_hbm.at[pl.ds(wid * k_per_w, k_per_w)], dst_v)
        pltpu.sync_copy(w_hbm.at[pl.ds(wid * k_per_w, k_per_w)], w_v)
        plsc.subcore_barrier()

        lane = lax.iota(jnp.int32, 16)

        def step(j, carry):
            pltpu.async_copy(table_hbm.at[src_v.at[j]], val_v, sem).wait()

            def scale_grp(g, cc):
                wv = w_v[j, pl.ds(16 * g, 16)]
                for l in range(16):
                    wl = jnp.full(
                        (16,),
                        jnp.sum(wv * (lane == l).astype(jnp.float32)))
                    e = 16 * g + l
                    for t in range(DIMS // 16):
                        x = val_v[e, pl.ds(t * 16, 16)]
                        val_v[e, pl.ds(t * 16, 16)] = x * wl
                return cc

            lax.fori_loop(0, CH // 16, scale_grp, 0)
            pltpu.sync_copy(val_v, acc_sh.at[dst_v.at[j]], add=True)
            return carry

        lax.fori_loop(0, k_per_w, step, 0)
        plsc.subcore_barrier()
        pltpu.sync_copy(acc_sh.at[pl.ds(s * rpt, rpt)],
                        out_hbm.at[pl.ds(c * n_pad + s * rpt, rpt)])

    return k(table, src2d, w2d, dst2d, zeros)


def _proj_body(ent_ref, wq_ref, out_ref):
    out_ref[...] = jnp.dot(ent_ref[...], wq_ref[...],
                           preferred_element_type=jnp.float32)


@jax.jit
def _tc_project(ent, wq):
    return pl.pallas_call(
        _proj_body,
        out_shape=jax.ShapeDtypeStruct(ent.shape, jnp.float32),
    )(ent, wq)


def _score_body(e_total, qh_ref, tail_ref, ridx_ref, rel_ref, wq_ref,
                ex_ref, uval_ref):
    i = pl.program_id(0)
    q = qh_ref[0]
    tail = tail_ref[0]
    ridx = ridx_ref[0, 0]
    onehot = (ridx[:, None] ==
              lax.broadcasted_iota(jnp.int32, (CB, N_REL), 1)).astype(
                  jnp.float32)
    relrow = jnp.dot(onehot, rel_ref[...], preferred_element_type=jnp.float32)
    tw = jnp.dot(tail, wq_ref[...], preferred_element_type=jnp.float32)
    prod = q * tw * relrow
    s0 = jnp.sum(prod[:, :D_K], axis=1) * (1.0 / math.sqrt(D_K))
    s1 = jnp.sum(prod[:, D_K:], axis=1) * (1.0 / math.sqrt(D_K))
    ex = jnp.exp(jnp.stack([s0, s1], axis=1))
    eid2 = i * CB + lax.broadcasted_iota(jnp.int32, (CB, 2), 0)
    ex = jnp.where(eid2 < e_total, ex, 0.0)
    ex_ref[0] = ex
    aexp = jnp.concatenate(
        [jnp.repeat(ex[:, 0:1], D_K, axis=1),
         jnp.repeat(ex[:, 1:2], D_K, axis=1)], axis=1)
    uval_ref[0] = tail * relrow * aexp


@functools.partial(jax.jit, static_argnames=("e_total",))
def _tc_scores(qh, tail, ridx3, rel, wq, *, e_total):
    gb = qh.shape[0]
    return pl.pallas_call(
        functools.partial(_score_body, e_total),
        grid=(gb,),
        in_specs=[
            pl.BlockSpec((1, CB, DIMS), lambda i: (i, 0, 0)),
            pl.BlockSpec((1, CB, DIMS), lambda i: (i, 0, 0)),
            pl.BlockSpec((1, 1, CB), lambda i: (i, 0, 0)),
            pl.BlockSpec((N_REL, DIMS), lambda i: (0, 0)),
            pl.BlockSpec((DIMS, DIMS), lambda i: (0, 0)),
        ],
        out_specs=[
            pl.BlockSpec((1, CB, 2), lambda i: (i, 0, 0)),
            pl.BlockSpec((1, CB, DIMS), lambda i: (i, 0, 0)),
        ],
        out_shape=[
            jax.ShapeDtypeStruct((gb, CB, 2), jnp.float32),
            jax.ShapeDtypeStruct((gb, CB, DIMS), jnp.float32),
        ],
    )(qh, tail, ridx3, rel, wq)


def _finalize_body(agg0_ref, agg1_ref, ss0_ref, ss1_ref, it0_ref, it1_ref,
                   ent_ref):
    agg = agg0_ref[pl.ds(0, N_ENT)] + agg1_ref[pl.ds(0, N_ENT)]
    ss = ss0_ref[pl.ds(0, N_ENT)] + ss1_ref[pl.ds(0, N_ENT)]
    den0 = ss[:, 0:1] + 1e-16
    den1 = ss[:, 1:2] + 1e-16
    agg = jnp.concatenate(
        [agg[:, :D_K] / den0, agg[:, D_K:] / den1], axis=1)
    n = jnp.sqrt(jnp.sum(agg * agg, axis=1, keepdims=True))
    agg = agg / jnp.maximum(n, 1e-12)
    ent_ref[...] = agg + it0_ref[pl.ds(0, N_ENT)] + it1_ref[pl.ds(0, N_ENT)]


@jax.jit
def _tc_finalize(agg0, agg1, ss0, ss1, it0, it1):
    return pl.pallas_call(
        _finalize_body,
        out_shape=jax.ShapeDtypeStruct((N_ENT, DIMS), jnp.float32),
    )(agg0, agg1, ss0, ss1, it0, it1)


def _add2_body(a_ref, b_ref, o_ref):
    o_ref[...] = a_ref[pl.ds(0, N_USR)] + b_ref[pl.ds(0, N_USR)]


@jax.jit
def _tc_add2(a, b):
    return pl.pallas_call(
        _add2_body,
        out_shape=jax.ShapeDtypeStruct((N_USR, DIMS), jnp.float32),
    )(a, b)


def _mean3_body(a_ref, b_ref, c_ref, o_ref):
    o_ref[...] = (a_ref[...] + b_ref[...] + c_ref[...]) * (1.0 / 3.0)


@jax.jit
def _tc_mean3(a, b, c):
    return pl.pallas_call(
        _mean3_body,
        out_shape=jax.ShapeDtypeStruct(a.shape, jnp.float32),
    )(a, b, c)


def _pad_edges(x, e_pad, fill=0):
    e = x.shape[0]
    if e == e_pad:
        return x
    return jnp.concatenate(
        [x, jnp.full((e_pad - e,) + x.shape[1:], fill, x.dtype)])


def kernel(layers_num, user_emb, entity_emb, inter_edge, inter_edge_w,
           edge_index, edge_type, relation_emb, W_Q):
    e_kg = edge_index.shape[1]
    e_int = inter_edge.shape[1]
    k_per_w = ((-(-e_kg // (NW * CH)) + 7) // 8) * 8
    e_pad = NW * k_per_w * CH
    ki_per_w = ((-(-e_int // (NW * CH)) + 7) // 8) * 8
    ei_pad = NW * ki_per_w * CH
    gb = e_pad // CB
    n_pad = ((max(N_ENT, N_USR) + 127) // 128) * 128

    head = _pad_edges(edge_index[0].astype(jnp.int32), e_pad)
    tail_i = _pad_edges(edge_index[1].astype(jnp.int32), e_pad)
    ridx = _pad_edges(((edge_type.astype(jnp.int32) - 1) % N_REL), e_pad)
    iu = _pad_edges(inter_edge[0].astype(jnp.int32), ei_pad)
    ii = _pad_edges(inter_edge[1].astype(jnp.int32), ei_pad)
    iw = _pad_edges(inter_edge_w.astype(jnp.float32), ei_pad)

    head2d = head.reshape(NW * k_per_w, CH)
    tail2d = tail_i.reshape(NW * k_per_w, CH)
    ridx3 = ridx.reshape(gb, 1, CB)
    iu2d = iu.reshape(NW * ki_per_w, CH)
    ii2d = ii.reshape(NW * ki_per_w, CH)
    iw2d = iw.reshape(NW * ki_per_w, CH)

    zeros128 = jnp.zeros((n_pad, DIMS), jnp.float32)

    usr = user_emb.astype(jnp.float32)
    ent = entity_emb.astype(jnp.float32)
    rel = relation_emb.astype(jnp.float32)
    wq = W_Q.astype(jnp.float32)

    user_embs = [usr]
    entity_embs = [ent]
    for _ in range(LAYERS):
        eq = _tc_project(ent, wq)
        qh = _sc_gather(eq, head2d, k_per_w=k_per_w, d=DIMS)
        tail_rows = _sc_gather(ent, tail2d, k_per_w=k_per_w, d=DIMS)
        qh3 = qh.reshape(gb, CB, DIMS)
        tail3 = tail_rows.reshape(gb, CB, DIMS)
        ex, uval = _tc_scores(qh3, tail3, ridx3, rel, wq, e_total=e_kg)
        agg_p = _sc_scatter_add(uval.reshape(e_pad, DIMS), head2d, zeros128,
                                k_per_w=k_per_w, n_pad=n_pad)
        ssum_p = _sc_ssum_scatter(ex.reshape(e_pad, 2), head2d, zeros128,
                                  k_per_w=k_per_w, n_pad=n_pad)
        user_p = _sc_gather_scale_scatter(ent, ii2d, iw2d, iu2d, zeros128,
                                          k_per_w=ki_per_w, n_pad=n_pad)
        item_p = _sc_gather_scale_scatter(usr, iu2d, iw2d, ii2d, zeros128,
                                          k_per_w=ki_per_w, n_pad=n_pad)
        ent = _tc_finalize(agg_p[:n_pad], agg_p[n_pad:],
                           ssum_p[:n_pad], ssum_p[n_pad:],
                           item_p[:n_pad], item_p[n_pad:])
        usr = _tc_add2(user_p[:n_pad], user_p[n_pad:])
        user_embs.append(usr)
        entity_embs.append(ent)

    user_out = _tc_mean3(*user_embs)
    entity_out = _tc_mean3(*entity_embs)
    return user_out, entity_out


# speedup vs baseline: 3.1429x; 1.0878x over previous
import functools
import math

import jax
import jax.numpy as jnp
from jax import lax
from jax.experimental import pallas as pl
from jax.experimental.pallas import tpu as pltpu
from jax.experimental.pallas import tpu_sc as plsc

N_ENT = 10000
N_USR = 10000
DIMS = 128
N_REL = 16
N_HEADS = 2
D_K = DIMS // N_HEADS
LAYERS = 2

NC = 2
NS = 16
NW = NC * NS
CH = 128
SB = 16
CB = 2048

_MESH = dict(core_axis_name="c", subcore_axis_name="s", num_cores=NC,
             num_subcores=NS)


@functools.partial(jax.jit, static_argnames=("k_per_w", "d"))
def _sc_gather(table, idx2d, *, k_per_w, d):
    b = NW * k_per_w * CH
    mesh = plsc.VectorSubcoreMesh(**_MESH)

    @functools.partial(
        pl.kernel,
        out_type=jax.ShapeDtypeStruct((b, d), jnp.float32),
        mesh=mesh,
        scratch_types=[
            pltpu.VMEM((k_per_w, CH), jnp.int32),
            pltpu.VMEM((CH, d), jnp.float32),
            pltpu.VMEM((CH, d), jnp.float32),
            pltpu.SemaphoreType.DMA,
            pltpu.SemaphoreType.DMA,
            pltpu.SemaphoreType.DMA,
            pltpu.SemaphoreType.DMA,
        ],
        compiler_params=pltpu.CompilerParams(needs_layout_passes=False),
    )
    def k(table_hbm, idx_hbm, out_hbm, idx_v, rows0, rows1, sg0, sg1,
          ss0, ss1):
        wid = lax.axis_index("s") * NC + lax.axis_index("c")
        pltpu.sync_copy(idx_hbm.at[pl.ds(wid * k_per_w, k_per_w)], idx_v)
        bufs = ((rows0, sg0, ss0), (rows1, sg1, ss1))

        def gather_start(j, buf, sg):
            pltpu.async_copy(table_hbm.at[idx_v.at[j]], buf, sg)

        def store_start(j, buf, ss):
            pltpu.async_copy(
                buf, out_hbm.at[pl.ds((wid * k_per_w + j) * CH, CH)], ss)

        gather_start(0, rows0, sg0)
        gather_start(1, rows1, sg1)

        def step(i, carry):
            jj = 2 * i
            for b in range(2):
                buf, sg, ss = bufs[b]
                pltpu.make_async_copy(table_hbm.at[idx_v.at[0]], buf,
                                      sg).wait()
                store_start(jj + b, buf, ss)
            for b in range(2):
                buf, sg, ss = bufs[b]

                @pl.when(jj + b + 2 < k_per_w)
                def _():
                    pltpu.make_async_copy(
                        buf, out_hbm.at[pl.ds(0, CH)], ss).wait()
                    gather_start(jj + b + 2, buf, sg)

            return carry

        lax.fori_loop(0, k_per_w // 2, step, 0)
        for b in range(2):
            buf, sg, ss = bufs[b]
            pltpu.make_async_copy(buf, out_hbm.at[pl.ds(0, CH)], ss).wait()

    return k(table, idx2d)


@functools.partial(jax.jit, static_argnames=("k_per_w", "n_pad"))
def _sc_scatter_add(values, idx2d, zeros, *, k_per_w, n_pad):
    mesh = plsc.VectorSubcoreMesh(**_MESH)
    rpt = n_pad // NS

    @functools.partial(
        pl.kernel,
        out_type=jax.ShapeDtypeStruct((NC * n_pad, DIMS), jnp.float32),
        mesh=mesh,
        scratch_types=[
            pltpu.VMEM((k_per_w, CH), jnp.int32),
            pltpu.VMEM((CH, DIMS), jnp.float32),
            pltpu.VMEM((CH, DIMS), jnp.float32),
            pltpu.VMEM_SHARED((n_pad, DIMS), jnp.float32),
            pltpu.SemaphoreType.DMA,
            pltpu.SemaphoreType.DMA,
            pltpu.SemaphoreType.DMA,
            pltpu.SemaphoreType.DMA,
        ],
        compiler_params=pltpu.CompilerParams(needs_layout_passes=False),
    )
    def k(val_hbm, idx_hbm, zero_hbm, out_hbm, idx_v, val0, val1, acc_sh,
          sl0, sl1, sc0, sc1):
        c = lax.axis_index("c")
        s = lax.axis_index("s")
        wid = s * NC + c
        pltpu.sync_copy(zero_hbm.at[pl.ds(s * rpt, rpt)],
                        acc_sh.at[pl.ds(s * rpt, rpt)])
        pltpu.sync_copy(idx_hbm.at[pl.ds(wid * k_per_w, k_per_w)], idx_v)
        plsc.subcore_barrier()
        bufs = ((val0, sl0, sc0), (val1, sl1, sc1))

        def load_start(j, buf, sl):
            pltpu.async_copy(
                val_hbm.at[pl.ds((wid * k_per_w + j) * CH, CH)], buf, sl)

        load_start(0, val0, sl0)
        load_start(1, val1, sl1)

        def step(i, carry):
            jj = 2 * i
            for b in range(2):
                buf, sl, sc = bufs[b]
                pltpu.make_async_copy(val_hbm.at[pl.ds(0, CH)], buf,
                                      sl).wait()
                pltpu.async_copy(buf, acc_sh.at[idx_v.at[jj + b]], sc,
                                 add=True)
            for b in range(2):
                buf, sl, sc = bufs[b]

                @pl.when(jj + b + 2 < k_per_w)
                def _():
                    pltpu.make_async_copy(buf, acc_sh.at[idx_v.at[0]],
                                          sc).wait()
                    load_start(jj + b + 2, buf, sl)

            return carry

        lax.fori_loop(0, k_per_w // 2, step, 0)
        for b in range(2):
            buf, sl, sc = bufs[b]
            pltpu.make_async_copy(buf, acc_sh.at[idx_v.at[0]], sc).wait()
        plsc.subcore_barrier()
        pltpu.sync_copy(acc_sh.at[pl.ds(s * rpt, rpt)],
                        out_hbm.at[pl.ds(c * n_pad + s * rpt, rpt)])

    return k(values, idx2d, zeros)


@functools.partial(jax.jit, static_argnames=("k_per_w", "n_pad"))
def _sc_ssum_scatter(ex2, idx2d, zeros, *, k_per_w, n_pad):
    mesh = plsc.VectorSubcoreMesh(**_MESH)
    rpt = n_pad // NS
    nex = k_per_w * CH * 2

    @functools.partial(
        pl.kernel,
        out_type=jax.ShapeDtypeStruct((NC * n_pad, DIMS), jnp.float32),
        mesh=mesh,
        scratch_types=[
            pltpu.VMEM((SB, CH), jnp.int32),
            pltpu.VMEM((SB, 2 * CH), jnp.float32),
            pltpu.VMEM((CH, DIMS), jnp.float32),
            pltpu.VMEM((CH, DIMS), jnp.float32),
            pltpu.VMEM_SHARED((n_pad, DIMS), jnp.float32),
            pltpu.SemaphoreType.DMA,
            pltpu.SemaphoreType.DMA,
        ],
        compiler_params=pltpu.CompilerParams(needs_layout_passes=False),
    )
    def k(ex_hbm, idx_hbm, zero_hbm, out_hbm, idx_v, ex_v, val0, val1,
          acc_sh, sc0, sc1):
        c = lax.axis_index("c")
        s = lax.axis_index("s")
        wid = s * NC + c
        pltpu.sync_copy(zero_hbm.at[pl.ds(s * rpt, rpt)],
                        acc_sh.at[pl.ds(s * rpt, rpt)])
        bufs = ((val0, sc0), (val1, sc1))

        def zrow(i, carry):
            for t in range(DIMS // 16):
                z = jnp.zeros((16,), jnp.float32)
                val0[i, pl.ds(t * 16, 16)] = z
                val1[i, pl.ds(t * 16, 16)] = z
            return carry

        lax.fori_loop(0, CH, zrow, 0)
        plsc.subcore_barrier()
        lane = lax.iota(jnp.int32, 16)
        c0 = (lane == 0).astype(jnp.float32)
        c1 = (lane == 1).astype(jnp.float32)

        def fill(j, buf):
            def fill_grp(g, cc):
                for half in range(2):
                    p = ex_v[j, pl.ds(32 * g + 16 * half, 16)]
                    for l in range(8):
                        e = 16 * g + 8 * half + l
                        v0 = jnp.sum(p * (lane == 2 * l).astype(jnp.float32))
                        v1 = jnp.sum(p * (lane == 2 * l + 1).astype(
                            jnp.float32))
                        buf[e, pl.ds(0, 16)] = (jnp.full((16,), v0) * c0 +
                                                jnp.full((16,), v1) * c1)
                return cc

            lax.fori_loop(0, CH // 16, fill_grp, 0)

        def seg(h, carry):
            @pl.when(h > 0)
            def _():
                for b in range(2):
                    buf, sc = bufs[b]
                    pltpu.make_async_copy(buf, acc_sh.at[idx_v.at[0]],
                                          sc).wait()

            base = wid * k_per_w + h * SB
            pltpu.sync_copy(idx_hbm.at[pl.ds(base, SB)], idx_v)
            pltpu.sync_copy(ex_hbm.at[pl.ds(base, SB)], ex_v)

            def step(i, cc):
                jj = 2 * i
                for b in range(2):
                    buf, sc = bufs[b]
                    j = jj + b

                    @pl.when(j >= 2)
                    def _():
                        pltpu.make_async_copy(buf, acc_sh.at[idx_v.at[0]],
                                              sc).wait()

                    fill(j, buf)
                    pltpu.async_copy(buf, acc_sh.at[idx_v.at[j]], sc,
                                     add=True)
                return cc

            lax.fori_loop(0, SB // 2, step, 0)
            return carry

        lax.fori_loop(0, k_per_w // SB, seg, 0)
        for b in range(2):
            buf, sc = bufs[b]
            pltpu.make_async_copy(buf, acc_sh.at[idx_v.at[0]], sc).wait()
        plsc.subcore_barrier()
        pltpu.sync_copy(acc_sh.at[pl.ds(s * rpt, rpt)],
                        out_hbm.at[pl.ds(c * n_pad + s * rpt, rpt)])

    return k(ex2.reshape(NW * k_per_w, 2 * CH), idx2d, zeros)


@functools.partial(jax.jit, static_argnames=("k_per_w", "n_pad"))
def _sc_gather_scale_scatter(table, src2d, w2d, dst2d, zeros, *,
                             k_per_w, n_pad):
    mesh = plsc.VectorSubcoreMesh(**_MESH)
    rpt = n_pad // NS

    @functools.partial(
        pl.kernel,
        out_type=jax.ShapeDtypeStruct((NC * n_pad, DIMS), jnp.float32),
        mesh=mesh,
        scratch_types=[
            pltpu.VMEM((SB, CH), jnp.int32),
            pltpu.VMEM((SB, CH), jnp.int32),
            pltpu.VMEM((SB, CH), jnp.float32),
            pltpu.VMEM((CH, DIMS), jnp.float32),
            pltpu.VMEM((CH, DIMS), jnp.float32),
            pltpu.VMEM_SHARED((n_pad, DIMS), jnp.float32),
            pltpu.SemaphoreType.DMA,
            pltpu.SemaphoreType.DMA,
            pltpu.SemaphoreType.DMA,
            pltpu.SemaphoreType.DMA,
        ],
        compiler_params=pltpu.CompilerParams(needs_layout_passes=False),
    )
    def k(table_hbm, src_hbm, w_hbm, dst_hbm, zero_hbm, out_hbm,
          src_v, dst_v, w_v, val0, val1, acc_sh, sg0, sg1, sc0, sc1):
        c = lax.axis_index("c")
        s = lax.axis_index("s")
        wid = s * NC + c
        pltpu.sync_copy(zero_hbm.at[pl.ds(s * rpt, rpt)],
                        acc_sh.at[pl.ds(s * rpt, rpt)])
        plsc.subcore_barrier()

        lane = lax.iota(jnp.int32, 16)
        bufs = ((val0, sg0, sc0), (val1, sg1, sc1))

        def gather_start(j, buf, sg):
            pltpu.async_copy(table_hbm.at[src_v.at[j]], buf, sg)

        def scale(j, buf):
            def scale_grp(g, cc):
                wv = w_v[j, pl.ds(16 * g, 16)]
                for l in range(16):
                    wl = jnp.full(
                        (16,),
                        jnp.sum(wv * (lane == l).astype(jnp.float32)))
                    e = 16 * g + l
                    for t in range(DIMS // 16):
                        x = buf[e, pl.ds(t * 16, 16)]
                        buf[e, pl.ds(t * 16, 16)] = x * wl
                return cc

            lax.fori_loop(0, CH // 16, scale_grp, 0)

        def seg(h, carry):
            @pl.when(h > 0)
            def _():
                for b in range(2):
                    buf, sg, sc = bufs[b]
                    pltpu.make_async_copy(buf, acc_sh.at[dst_v.at[0]],
                                          sc).wait()

            base = wid * k_per_w + h * SB
            pltpu.sync_copy(src_hbm.at[pl.ds(base, SB)], src_v)
            pltpu.sync_copy(dst_hbm.at[pl.ds(base, SB)], dst_v)
            pltpu.sync_copy(w_hbm.at[pl.ds(base, SB)], w_v)
            gather_start(0, val0, sg0)
            gather_start(1, val1, sg1)

            def step(i, cc):
                jj = 2 * i
                for b in range(2):
                    buf, sg, sc = bufs[b]
                    pltpu.make_async_copy(table_hbm.at[src_v.at[0]], buf,
                                          sg).wait()
                    scale(jj + b, buf)
                    pltpu.async_copy(buf, acc_sh.at[dst_v.at[jj + b]], sc,
                                     add=True)
                for b in range(2):
                    buf, sg, sc = bufs[b]

                    @pl.when(jj + b + 2 < SB)
                    def _():
                        pltpu.make_async_copy(buf, acc_sh.at[dst_v.at[0]],
                                              sc).wait()
                        gather_start(jj + b + 2, buf, sg)

                return cc

            lax.fori_loop(0, SB // 2, step, 0)
            return carry

        lax.fori_loop(0, k_per_w // SB, seg, 0)
        for b in range(2):
            buf, sg, sc = bufs[b]
            pltpu.make_async_copy(buf, acc_sh.at[dst_v.at[0]], sc).wait()
        plsc.subcore_barrier()
        pltpu.sync_copy(acc_sh.at[pl.ds(s * rpt, rpt)],
                        out_hbm.at[pl.ds(c * n_pad + s * rpt, rpt)])

    return k(table, src2d, w2d, dst2d, zeros)


def _proj_body(ent_ref, wq_ref, out_ref):
    out_ref[...] = jnp.dot(ent_ref[...], wq_ref[...],
                           preferred_element_type=jnp.float32)


@jax.jit
def _tc_project(ent, wq):
    return pl.pallas_call(
        _proj_body,
        out_shape=jax.ShapeDtypeStruct(ent.shape, jnp.float32),
    )(ent, wq)


def _score_body(e_total, qh_ref, tail_ref, ridx_ref, rel_ref, wq_ref,
                ex_ref, uval_ref):
    i = pl.program_id(0)
    q = qh_ref[0]
    tail = tail_ref[0]
    ridx = ridx_ref[0, 0]
    onehot = (ridx[:, None] ==
              lax.broadcasted_iota(jnp.int32, (CB, N_REL), 1)).astype(
                  jnp.float32)
    relrow = jnp.dot(onehot, rel_ref[...], preferred_element_type=jnp.float32)
    tw = jnp.dot(tail, wq_ref[...], preferred_element_type=jnp.float32)
    prod = q * tw * relrow
    s0 = jnp.sum(prod[:, :D_K], axis=1) * (1.0 / math.sqrt(D_K))
    s1 = jnp.sum(prod[:, D_K:], axis=1) * (1.0 / math.sqrt(D_K))
    ex = jnp.exp(jnp.stack([s0, s1], axis=1))
    eid2 = i * CB + lax.broadcasted_iota(jnp.int32, (CB, 2), 0)
    ex = jnp.where(eid2 < e_total, ex, 0.0)
    ex_ref[0] = ex
    aexp = jnp.concatenate(
        [jnp.repeat(ex[:, 0:1], D_K, axis=1),
         jnp.repeat(ex[:, 1:2], D_K, axis=1)], axis=1)
    uval_ref[0] = tail * relrow * aexp


@functools.partial(jax.jit, static_argnames=("e_total",))
def _tc_scores(qh, tail, ridx3, rel, wq, *, e_total):
    gb = qh.shape[0]
    return pl.pallas_call(
        functools.partial(_score_body, e_total),
        grid=(gb,),
        in_specs=[
            pl.BlockSpec((1, CB, DIMS), lambda i: (i, 0, 0)),
            pl.BlockSpec((1, CB, DIMS), lambda i: (i, 0, 0)),
            pl.BlockSpec((1, 1, CB), lambda i: (i, 0, 0)),
            pl.BlockSpec((N_REL, DIMS), lambda i: (0, 0)),
            pl.BlockSpec((DIMS, DIMS), lambda i: (0, 0)),
        ],
        out_specs=[
            pl.BlockSpec((1, CB, 2), lambda i: (i, 0, 0)),
            pl.BlockSpec((1, CB, DIMS), lambda i: (i, 0, 0)),
        ],
        out_shape=[
            jax.ShapeDtypeStruct((gb, CB, 2), jnp.float32),
            jax.ShapeDtypeStruct((gb, CB, DIMS), jnp.float32),
        ],
    )(qh, tail, ridx3, rel, wq)


def _finalize_body(agg0_ref, agg1_ref, ss0_ref, ss1_ref, it0_ref, it1_ref,
                   ent_ref):
    agg = agg0_ref[pl.ds(0, N_ENT)] + agg1_ref[pl.ds(0, N_ENT)]
    ss = ss0_ref[pl.ds(0, N_ENT)] + ss1_ref[pl.ds(0, N_ENT)]
    den0 = ss[:, 0:1] + 1e-16
    den1 = ss[:, 1:2] + 1e-16
    agg = jnp.concatenate(
        [agg[:, :D_K] / den0, agg[:, D_K:] / den1], axis=1)
    n = jnp.sqrt(jnp.sum(agg * agg, axis=1, keepdims=True))
    agg = agg / jnp.maximum(n, 1e-12)
    ent_ref[...] = agg + it0_ref[pl.ds(0, N_ENT)] + it1_ref[pl.ds(0, N_ENT)]


@jax.jit
def _tc_finalize(agg0, agg1, ss0, ss1, it0, it1):
    return pl.pallas_call(
        _finalize_body,
        out_shape=jax.ShapeDtypeStruct((N_ENT, DIMS), jnp.float32),
    )(agg0, agg1, ss0, ss1, it0, it1)


def _add2_body(a_ref, b_ref, o_ref):
    o_ref[...] = a_ref[pl.ds(0, N_USR)] + b_ref[pl.ds(0, N_USR)]


@jax.jit
def _tc_add2(a, b):
    return pl.pallas_call(
        _add2_body,
        out_shape=jax.ShapeDtypeStruct((N_USR, DIMS), jnp.float32),
    )(a, b)


def _mean3_body(a_ref, b_ref, c_ref, o_ref):
    o_ref[...] = (a_ref[...] + b_ref[...] + c_ref[...]) * (1.0 / 3.0)


@jax.jit
def _tc_mean3(a, b, c):
    return pl.pallas_call(
        _mean3_body,
        out_shape=jax.ShapeDtypeStruct(a.shape, jnp.float32),
    )(a, b, c)


def _pad_edges(x, e_pad, fill=0):
    e = x.shape[0]
    if e == e_pad:
        return x
    return jnp.concatenate(
        [x, jnp.full((e_pad - e,) + x.shape[1:], fill, x.dtype)])


def kernel(layers_num, user_emb, entity_emb, inter_edge, inter_edge_w,
           edge_index, edge_type, relation_emb, W_Q):
    e_kg = edge_index.shape[1]
    e_int = inter_edge.shape[1]
    k_per_w = ((-(-e_kg // (NW * CH)) + 7) // 8) * 8
    e_pad = NW * k_per_w * CH
    ki_per_w = ((-(-e_int // (NW * CH)) + 7) // 8) * 8
    ei_pad = NW * ki_per_w * CH
    gb = e_pad // CB
    n_pad = ((max(N_ENT, N_USR) + 127) // 128) * 128

    head = _pad_edges(edge_index[0].astype(jnp.int32), e_pad)
    tail_i = _pad_edges(edge_index[1].astype(jnp.int32), e_pad)
    ridx = _pad_edges(((edge_type.astype(jnp.int32) - 1) % N_REL), e_pad)
    iu = _pad_edges(inter_edge[0].astype(jnp.int32), ei_pad)
    ii = _pad_edges(inter_edge[1].astype(jnp.int32), ei_pad)
    iw = _pad_edges(inter_edge_w.astype(jnp.float32), ei_pad)

    head2d = head.reshape(NW * k_per_w, CH)
    tail2d = tail_i.reshape(NW * k_per_w, CH)
    ridx3 = ridx.reshape(gb, 1, CB)
    iu2d = iu.reshape(NW * ki_per_w, CH)
    ii2d = ii.reshape(NW * ki_per_w, CH)
    iw2d = iw.reshape(NW * ki_per_w, CH)

    zeros128 = jnp.zeros((n_pad, DIMS), jnp.float32)

    usr = user_emb.astype(jnp.float32)
    ent = entity_emb.astype(jnp.float32)
    rel = relation_emb.astype(jnp.float32)
    wq = W_Q.astype(jnp.float32)

    user_embs = [usr]
    entity_embs = [ent]
    for _ in range(LAYERS):
        eq = _tc_project(ent, wq)
        qh = _sc_gather(eq, head2d, k_per_w=k_per_w, d=DIMS)
        tail_rows = _sc_gather(ent, tail2d, k_per_w=k_per_w, d=DIMS)
        qh3 = qh.reshape(gb, CB, DIMS)
        tail3 = tail_rows.reshape(gb, CB, DIMS)
        ex, uval = _tc_scores(qh3, tail3, ridx3, rel, wq, e_total=e_kg)
        agg_p = _sc_scatter_add(uval.reshape(e_pad, DIMS), head2d, zeros128,
                                k_per_w=k_per_w, n_pad=n_pad)
        ssum_p = _sc_ssum_scatter(ex.reshape(e_pad, 2), head2d, zeros128,
                                  k_per_w=k_per_w, n_pad=n_pad)
        user_p = _sc_gather_scale_scatter(ent, ii2d, iw2d, iu2d, zeros128,
                                          k_per_w=ki_per_w, n_pad=n_pad)
        item_p = _sc_gather_scale_scatter(usr, iu2d, iw2d, ii2d, zeros128,
                                          k_per_w=ki_per_w, n_pad=n_pad)
        ent = _tc_finalize(agg_p[:n_pad], agg_p[n_pad:],
                           ssum_p[:n_pad], ssum_p[n_pad:],
                           item_p[:n_pad], item_p[n_pad:])
        usr = _tc_add2(user_p[:n_pad], user_p[n_pad:])
        user_embs.append(usr)
        entity_embs.append(ent)

    user_out = _tc_mean3(*user_embs)
    entity_out = _tc_mean3(*entity_embs)
    return user_out, entity_out


# speedup vs baseline: 4.5191x; 1.4379x over previous
import functools
import math

import jax
import jax.numpy as jnp
from jax import lax
from jax.experimental import pallas as pl
from jax.experimental.pallas import tpu as pltpu
from jax.experimental.pallas import tpu_sc as plsc

N_ENT = 10000
N_USR = 10000
DIMS = 128
N_REL = 16
N_HEADS = 2
D_K = DIMS // N_HEADS
LAYERS = 2

NC = 2
NS = 16
NW = NC * NS
CH = 128
SB = 16
CB = 2048

_MESH = dict(core_axis_name="c", subcore_axis_name="s", num_cores=NC,
             num_subcores=NS)


@functools.partial(jax.jit, static_argnames=("k_per_w", "d"))
def _sc_gather(table, idx2d, *, k_per_w, d):
    b = NW * k_per_w * CH
    v = table.shape[0]
    rv = (v // NS) // 8 * 8
    rv_last = v - rv * (NS - 1)
    mesh = plsc.VectorSubcoreMesh(**_MESH)

    @functools.partial(
        pl.kernel,
        out_type=jax.ShapeDtypeStruct((b, d), jnp.float32),
        mesh=mesh,
        scratch_types=[
            pltpu.VMEM((k_per_w, CH), jnp.int32),
            pltpu.VMEM((CH, d), jnp.float32),
            pltpu.VMEM((CH, d), jnp.float32),
            pltpu.VMEM_SHARED((v, d), jnp.float32),
            pltpu.SemaphoreType.DMA,
            pltpu.SemaphoreType.DMA,
            pltpu.SemaphoreType.DMA,
            pltpu.SemaphoreType.DMA,
        ],
        compiler_params=pltpu.CompilerParams(needs_layout_passes=False),
    )
    def k(table_hbm, idx_hbm, out_hbm, idx_v, rows0, rows1, tab_sh, sg0,
          sg1, ss0, ss1):
        s = lax.axis_index("s")
        wid = s * NC + lax.axis_index("c")

        @pl.when(s < NS - 1)
        def _():
            pltpu.sync_copy(table_hbm.at[pl.ds(s * rv, rv)],
                            tab_sh.at[pl.ds(s * rv, rv)])

        @pl.when(s == NS - 1)
        def _():
            pltpu.sync_copy(table_hbm.at[pl.ds((NS - 1) * rv, rv_last)],
                            tab_sh.at[pl.ds((NS - 1) * rv, rv_last)])

        pltpu.sync_copy(idx_hbm.at[pl.ds(wid * k_per_w, k_per_w)], idx_v)
        plsc.subcore_barrier()
        bufs = ((rows0, sg0, ss0), (rows1, sg1, ss1))

        def gather_start(j, buf, sg):
            pltpu.async_copy(tab_sh.at[idx_v.at[j]], buf, sg)

        def store_start(j, buf, ss):
            pltpu.async_copy(
                buf, out_hbm.at[pl.ds((wid * k_per_w + j) * CH, CH)], ss)

        gather_start(0, rows0, sg0)
        gather_start(1, rows1, sg1)

        def step(i, carry):
            jj = 2 * i
            for b in range(2):
                buf, sg, ss = bufs[b]
                pltpu.make_async_copy(table_hbm.at[idx_v.at[0]], buf,
                                      sg).wait()
                store_start(jj + b, buf, ss)
            for b in range(2):
                buf, sg, ss = bufs[b]

                @pl.when(jj + b + 2 < k_per_w)
                def _():
                    pltpu.make_async_copy(
                        buf, out_hbm.at[pl.ds(0, CH)], ss).wait()
                    gather_start(jj + b + 2, buf, sg)

            return carry

        lax.fori_loop(0, k_per_w // 2, step, 0)
        for b in range(2):
            buf, sg, ss = bufs[b]
            pltpu.make_async_copy(buf, out_hbm.at[pl.ds(0, CH)], ss).wait()

    return k(table, idx2d)


@functools.partial(jax.jit, static_argnames=("k_per_w", "n_pad"))
def _sc_scatter_add(values, idx2d, zeros, *, k_per_w, n_pad):
    mesh = plsc.VectorSubcoreMesh(**_MESH)
    rpt = n_pad // NS

    @functools.partial(
        pl.kernel,
        out_type=jax.ShapeDtypeStruct((NC * n_pad, DIMS), jnp.float32),
        mesh=mesh,
        scratch_types=[
            pltpu.VMEM((k_per_w, CH), jnp.int32),
            pltpu.VMEM((CH, DIMS), jnp.float32),
            pltpu.VMEM((CH, DIMS), jnp.float32),
            pltpu.VMEM_SHARED((n_pad, DIMS), jnp.float32),
            pltpu.SemaphoreType.DMA,
            pltpu.SemaphoreType.DMA,
            pltpu.SemaphoreType.DMA,
            pltpu.SemaphoreType.DMA,
        ],
        compiler_params=pltpu.CompilerParams(needs_layout_passes=False),
    )
    def k(val_hbm, idx_hbm, zero_hbm, out_hbm, idx_v, val0, val1, acc_sh,
          sl0, sl1, sc0, sc1):
        c = lax.axis_index("c")
        s = lax.axis_index("s")
        wid = s * NC + c
        pltpu.sync_copy(zero_hbm.at[pl.ds(s * rpt, rpt)],
                        acc_sh.at[pl.ds(s * rpt, rpt)])
        pltpu.sync_copy(idx_hbm.at[pl.ds(wid * k_per_w, k_per_w)], idx_v)
        plsc.subcore_barrier()
        bufs = ((val0, sl0, sc0), (val1, sl1, sc1))

        def load_start(j, buf, sl):
            pltpu.async_copy(
                val_hbm.at[pl.ds((wid * k_per_w + j) * CH, CH)], buf, sl)

        load_start(0, val0, sl0)
        load_start(1, val1, sl1)

        def step(i, carry):
            jj = 2 * i
            for b in range(2):
                buf, sl, sc = bufs[b]
                pltpu.make_async_copy(val_hbm.at[pl.ds(0, CH)], buf,
                                      sl).wait()
                pltpu.async_copy(buf, acc_sh.at[idx_v.at[jj + b]], sc,
                                 add=True)
            for b in range(2):
                buf, sl, sc = bufs[b]

                @pl.when(jj + b + 2 < k_per_w)
                def _():
                    pltpu.make_async_copy(buf, acc_sh.at[idx_v.at[0]],
                                          sc).wait()
                    load_start(jj + b + 2, buf, sl)

            return carry

        lax.fori_loop(0, k_per_w // 2, step, 0)
        for b in range(2):
            buf, sl, sc = bufs[b]
            pltpu.make_async_copy(buf, acc_sh.at[idx_v.at[0]], sc).wait()
        plsc.subcore_barrier()
        pltpu.sync_copy(acc_sh.at[pl.ds(s * rpt, rpt)],
                        out_hbm.at[pl.ds(c * n_pad + s * rpt, rpt)])

    return k(values, idx2d, zeros)


@functools.partial(jax.jit, static_argnames=("k_per_w", "n_pad"))
def _sc_ssum_scatter(ex2, idx2d, zeros, *, k_per_w, n_pad):
    mesh = plsc.VectorSubcoreMesh(**_MESH)
    rpt = n_pad // NS
    nex = k_per_w * CH * 2

    @functools.partial(
        pl.kernel,
        out_type=jax.ShapeDtypeStruct((NC * n_pad, DIMS), jnp.float32),
        mesh=mesh,
        scratch_types=[
            pltpu.VMEM((SB, CH), jnp.int32),
            pltpu.VMEM((SB, 2 * CH), jnp.float32),
            pltpu.VMEM((CH, DIMS), jnp.float32),
            pltpu.VMEM((CH, DIMS), jnp.float32),
            pltpu.VMEM_SHARED((n_pad, DIMS), jnp.float32),
            pltpu.SemaphoreType.DMA,
            pltpu.SemaphoreType.DMA,
        ],
        compiler_params=pltpu.CompilerParams(needs_layout_passes=False),
    )
    def k(ex_hbm, idx_hbm, zero_hbm, out_hbm, idx_v, ex_v, val0, val1,
          acc_sh, sc0, sc1):
        c = lax.axis_index("c")
        s = lax.axis_index("s")
        wid = s * NC + c
        pltpu.sync_copy(zero_hbm.at[pl.ds(s * rpt, rpt)],
                        acc_sh.at[pl.ds(s * rpt, rpt)])
        bufs = ((val0, sc0), (val1, sc1))

        def zrow(i, carry):
            for t in range(DIMS // 16):
                z = jnp.zeros((16,), jnp.float32)
                val0[i, pl.ds(t * 16, 16)] = z
                val1[i, pl.ds(t * 16, 16)] = z
            return carry

        lax.fori_loop(0, CH, zrow, 0)
        plsc.subcore_barrier()
        lane = lax.iota(jnp.int32, 16)
        c0 = (lane == 0).astype(jnp.float32)
        c1 = (lane == 1).astype(jnp.float32)

        def fill(j, buf):
            def fill_grp(g, cc):
                for half in range(2):
                    p = ex_v[j, pl.ds(32 * g + 16 * half, 16)]
                    for l in range(8):
                        e = 16 * g + 8 * half + l
                        v0 = jnp.sum(p * (lane == 2 * l).astype(jnp.float32))
                        v1 = jnp.sum(p * (lane == 2 * l + 1).astype(
                            jnp.float32))
                        buf[e, pl.ds(0, 16)] = (jnp.full((16,), v0) * c0 +
                                                jnp.full((16,), v1) * c1)
                return cc

            lax.fori_loop(0, CH // 16, fill_grp, 0)

        def seg(h, carry):
            @pl.when(h > 0)
            def _():
                for b in range(2):
                    buf, sc = bufs[b]
                    pltpu.make_async_copy(buf, acc_sh.at[idx_v.at[0]],
                                          sc).wait()

            base = wid * k_per_w + h * SB
            pltpu.sync_copy(idx_hbm.at[pl.ds(base, SB)], idx_v)
            pltpu.sync_copy(ex_hbm.at[pl.ds(base, SB)], ex_v)

            def step(i, cc):
                jj = 2 * i
                for b in range(2):
                    buf, sc = bufs[b]
                    j = jj + b

                    @pl.when(j >= 2)
                    def _():
                        pltpu.make_async_copy(buf, acc_sh.at[idx_v.at[0]],
                                              sc).wait()

                    fill(j, buf)
                    pltpu.async_copy(buf, acc_sh.at[idx_v.at[j]], sc,
                                     add=True)
                return cc

            lax.fori_loop(0, SB // 2, step, 0)
            return carry

        lax.fori_loop(0, k_per_w // SB, seg, 0)
        for b in range(2):
            buf, sc = bufs[b]
            pltpu.make_async_copy(buf, acc_sh.at[idx_v.at[0]], sc).wait()
        plsc.subcore_barrier()
        pltpu.sync_copy(acc_sh.at[pl.ds(s * rpt, rpt)],
                        out_hbm.at[pl.ds(c * n_pad + s * rpt, rpt)])

    return k(ex2.reshape(NW * k_per_w, 2 * CH), idx2d, zeros)


@functools.partial(jax.jit, static_argnames=("k_per_w", "n_pad"))
def _sc_gather_scale_scatter(table, src2d, w2d, dst2d, zeros, *,
                             k_per_w, n_pad):
    mesh = plsc.VectorSubcoreMesh(**_MESH)
    rpt = n_pad // NS

    @functools.partial(
        pl.kernel,
        out_type=jax.ShapeDtypeStruct((NC * n_pad, DIMS), jnp.float32),
        mesh=mesh,
        scratch_types=[
            pltpu.VMEM((SB, CH), jnp.int32),
            pltpu.VMEM((SB, CH), jnp.int32),
            pltpu.VMEM((SB, CH), jnp.float32),
            pltpu.VMEM((CH, DIMS), jnp.float32),
            pltpu.VMEM((CH, DIMS), jnp.float32),
            pltpu.VMEM_SHARED((n_pad, DIMS), jnp.float32),
            pltpu.SemaphoreType.DMA,
            pltpu.SemaphoreType.DMA,
            pltpu.SemaphoreType.DMA,
            pltpu.SemaphoreType.DMA,
        ],
        compiler_params=pltpu.CompilerParams(needs_layout_passes=False),
    )
    def k(table_hbm, src_hbm, w_hbm, dst_hbm, zero_hbm, out_hbm,
          src_v, dst_v, w_v, val0, val1, acc_sh, sg0, sg1, sc0, sc1):
        c = lax.axis_index("c")
        s = lax.axis_index("s")
        wid = s * NC + c
        pltpu.sync_copy(zero_hbm.at[pl.ds(s * rpt, rpt)],
                        acc_sh.at[pl.ds(s * rpt, rpt)])
        plsc.subcore_barrier()

        lane = lax.iota(jnp.int32, 16)
        bufs = ((val0, sg0, sc0), (val1, sg1, sc1))

        def gather_start(j, buf, sg):
            pltpu.async_copy(table_hbm.at[src_v.at[j]], buf, sg)

        def scale(j, buf):
            def scale_grp(g, cc):
                wv = w_v[j, pl.ds(16 * g, 16)]
                for l in range(16):
                    wl = jnp.full(
                        (16,),
                        jnp.sum(wv * (lane == l).astype(jnp.float32)))
                    e = 16 * g + l
                    for t in range(DIMS // 16):
                        x = buf[e, pl.ds(t * 16, 16)]
                        buf[e, pl.ds(t * 16, 16)] = x * wl
                return cc

            lax.fori_loop(0, CH // 16, scale_grp, 0)

        def seg(h, carry):
            @pl.when(h > 0)
            def _():
                for b in range(2):
                    buf, sg, sc = bufs[b]
                    pltpu.make_async_copy(buf, acc_sh.at[dst_v.at[0]],
                                          sc).wait()

            base = wid * k_per_w + h * SB
            pltpu.sync_copy(src_hbm.at[pl.ds(base, SB)], src_v)
            pltpu.sync_copy(dst_hbm.at[pl.ds(base, SB)], dst_v)
            pltpu.sync_copy(w_hbm.at[pl.ds(base, SB)], w_v)
            gather_start(0, val0, sg0)
            gather_start(1, val1, sg1)

            def step(i, cc):
                jj = 2 * i
                for b in range(2):
                    buf, sg, sc = bufs[b]
                    pltpu.make_async_copy(table_hbm.at[src_v.at[0]], buf,
                                          sg).wait()
                    scale(jj + b, buf)
                    pltpu.async_copy(buf, acc_sh.at[dst_v.at[jj + b]], sc,
                                     add=True)
                for b in range(2):
                    buf, sg, sc = bufs[b]

                    @pl.when(jj + b + 2 < SB)
                    def _():
                        pltpu.make_async_copy(buf, acc_sh.at[dst_v.at[0]],
                                              sc).wait()
                        gather_start(jj + b + 2, buf, sg)

                return cc

            lax.fori_loop(0, SB // 2, step, 0)
            return carry

        lax.fori_loop(0, k_per_w // SB, seg, 0)
        for b in range(2):
            buf, sg, sc = bufs[b]
            pltpu.make_async_copy(buf, acc_sh.at[dst_v.at[0]], sc).wait()
        plsc.subcore_barrier()
        pltpu.sync_copy(acc_sh.at[pl.ds(s * rpt, rpt)],
                        out_hbm.at[pl.ds(c * n_pad + s * rpt, rpt)])

    return k(table, src2d, w2d, dst2d, zeros)


def _proj_body(ent_ref, wq_ref, out_ref):
    out_ref[...] = jnp.dot(ent_ref[...], wq_ref[...],
                           preferred_element_type=jnp.float32)


@jax.jit
def _tc_project(ent, wq):
    return pl.pallas_call(
        _proj_body,
        out_shape=jax.ShapeDtypeStruct(ent.shape, jnp.float32),
    )(ent, wq)


def _score_body(e_total, qh_ref, tail_ref, ridx_ref, rel_ref, wq_ref,
                ex_ref, uval_ref):
    i = pl.program_id(0)
    q = qh_ref[0]
    tail = tail_ref[0]
    ridx = ridx_ref[0, 0]
    onehot = (ridx[:, None] ==
              lax.broadcasted_iota(jnp.int32, (CB, N_REL), 1)).astype(
                  jnp.float32)
    relrow = jnp.dot(onehot, rel_ref[...], preferred_element_type=jnp.float32)
    tw = jnp.dot(tail, wq_ref[...], preferred_element_type=jnp.float32)
    prod = q * tw * relrow
    s0 = jnp.sum(prod[:, :D_K], axis=1) * (1.0 / math.sqrt(D_K))
    s1 = jnp.sum(prod[:, D_K:], axis=1) * (1.0 / math.sqrt(D_K))
    ex = jnp.exp(jnp.stack([s0, s1], axis=1))
    eid2 = i * CB + lax.broadcasted_iota(jnp.int32, (CB, 2), 0)
    ex = jnp.where(eid2 < e_total, ex, 0.0)
    ex_ref[0] = ex
    aexp = jnp.concatenate(
        [jnp.repeat(ex[:, 0:1], D_K, axis=1),
         jnp.repeat(ex[:, 1:2], D_K, axis=1)], axis=1)
    uval_ref[0] = tail * relrow * aexp


@functools.partial(jax.jit, static_argnames=("e_total",))
def _tc_scores(qh, tail, ridx3, rel, wq, *, e_total):
    gb = qh.shape[0]
    return pl.pallas_call(
        functools.partial(_score_body, e_total),
        grid=(gb,),
        in_specs=[
            pl.BlockSpec((1, CB, DIMS), lambda i: (i, 0, 0)),
            pl.BlockSpec((1, CB, DIMS), lambda i: (i, 0, 0)),
            pl.BlockSpec((1, 1, CB), lambda i: (i, 0, 0)),
            pl.BlockSpec((N_REL, DIMS), lambda i: (0, 0)),
            pl.BlockSpec((DIMS, DIMS), lambda i: (0, 0)),
        ],
        out_specs=[
            pl.BlockSpec((1, CB, 2), lambda i: (i, 0, 0)),
            pl.BlockSpec((1, CB, DIMS), lambda i: (i, 0, 0)),
        ],
        out_shape=[
            jax.ShapeDtypeStruct((gb, CB, 2), jnp.float32),
            jax.ShapeDtypeStruct((gb, CB, DIMS), jnp.float32),
        ],
    )(qh, tail, ridx3, rel, wq)


def _finalize_body(agg0_ref, agg1_ref, ss0_ref, ss1_ref, it0_ref, it1_ref,
                   ent_ref):
    agg = agg0_ref[pl.ds(0, N_ENT)] + agg1_ref[pl.ds(0, N_ENT)]
    ss = ss0_ref[pl.ds(0, N_ENT)] + ss1_ref[pl.ds(0, N_ENT)]
    den0 = ss[:, 0:1] + 1e-16
    den1 = ss[:, 1:2] + 1e-16
    agg = jnp.concatenate(
        [agg[:, :D_K] / den0, agg[:, D_K:] / den1], axis=1)
    n = jnp.sqrt(jnp.sum(agg * agg, axis=1, keepdims=True))
    agg = agg / jnp.maximum(n, 1e-12)
    ent_ref[...] = agg + it0_ref[pl.ds(0, N_ENT)] + it1_ref[pl.ds(0, N_ENT)]


@jax.jit
def _tc_finalize(agg0, agg1, ss0, ss1, it0, it1):
    return pl.pallas_call(
        _finalize_body,
        out_shape=jax.ShapeDtypeStruct((N_ENT, DIMS), jnp.float32),
    )(agg0, agg1, ss0, ss1, it0, it1)


def _add2_body(a_ref, b_ref, o_ref):
    o_ref[...] = a_ref[pl.ds(0, N_USR)] + b_ref[pl.ds(0, N_USR)]


@jax.jit
def _tc_add2(a, b):
    return pl.pallas_call(
        _add2_body,
        out_shape=jax.ShapeDtypeStruct((N_USR, DIMS), jnp.float32),
    )(a, b)


def _mean3_body(a_ref, b_ref, c_ref, o_ref):
    o_ref[...] = (a_ref[...] + b_ref[...] + c_ref[...]) * (1.0 / 3.0)


@jax.jit
def _tc_mean3(a, b, c):
    return pl.pallas_call(
        _mean3_body,
        out_shape=jax.ShapeDtypeStruct(a.shape, jnp.float32),
    )(a, b, c)


def _pad_edges(x, e_pad, fill=0):
    e = x.shape[0]
    if e == e_pad:
        return x
    return jnp.concatenate(
        [x, jnp.full((e_pad - e,) + x.shape[1:], fill, x.dtype)])


def kernel(layers_num, user_emb, entity_emb, inter_edge, inter_edge_w,
           edge_index, edge_type, relation_emb, W_Q):
    e_kg = edge_index.shape[1]
    e_int = inter_edge.shape[1]
    k_per_w = ((-(-e_kg // (NW * CH)) + 7) // 8) * 8
    e_pad = NW * k_per_w * CH
    ki_per_w = ((-(-e_int // (NW * CH)) + 7) // 8) * 8
    ei_pad = NW * ki_per_w * CH
    gb = e_pad // CB
    n_pad = ((max(N_ENT, N_USR) + 127) // 128) * 128

    head = _pad_edges(edge_index[0].astype(jnp.int32), e_pad)
    tail_i = _pad_edges(edge_index[1].astype(jnp.int32), e_pad)
    ridx = _pad_edges(((edge_type.astype(jnp.int32) - 1) % N_REL), e_pad)
    iu = _pad_edges(inter_edge[0].astype(jnp.int32), ei_pad)
    ii = _pad_edges(inter_edge[1].astype(jnp.int32), ei_pad)
    iw = _pad_edges(inter_edge_w.astype(jnp.float32), ei_pad)

    head2d = head.reshape(NW * k_per_w, CH)
    tail2d = tail_i.reshape(NW * k_per_w, CH)
    ridx3 = ridx.reshape(gb, 1, CB)
    iu2d = iu.reshape(NW * ki_per_w, CH)
    ii2d = ii.reshape(NW * ki_per_w, CH)
    iw2d = iw.reshape(NW * ki_per_w, CH)

    zeros128 = jnp.zeros((n_pad, DIMS), jnp.float32)

    usr = user_emb.astype(jnp.float32)
    ent = entity_emb.astype(jnp.float32)
    rel = relation_emb.astype(jnp.float32)
    wq = W_Q.astype(jnp.float32)

    user_embs = [usr]
    entity_embs = [ent]
    for _ in range(LAYERS):
        eq = _tc_project(ent, wq)
        qh = _sc_gather(eq, head2d, k_per_w=k_per_w, d=DIMS)
        tail_rows = _sc_gather(ent, tail2d, k_per_w=k_per_w, d=DIMS)
        qh3 = qh.reshape(gb, CB, DIMS)
        tail3 = tail_rows.reshape(gb, CB, DIMS)
        ex, uval = _tc_scores(qh3, tail3, ridx3, rel, wq, e_total=e_kg)
        agg_p = _sc_scatter_add(uval.reshape(e_pad, DIMS), head2d, zeros128,
                                k_per_w=k_per_w, n_pad=n_pad)
        ssum_p = _sc_ssum_scatter(ex.reshape(e_pad, 2), head2d, zeros128,
                                  k_per_w=k_per_w, n_pad=n_pad)
        user_p = _sc_gather_scale_scatter(ent, ii2d, iw2d, iu2d, zeros128,
                                          k_per_w=ki_per_w, n_pad=n_pad)
        item_p = _sc_gather_scale_scatter(usr, iu2d, iw2d, ii2d, zeros128,
                                          k_per_w=ki_per_w, n_pad=n_pad)
        ent = _tc_finalize(agg_p[:n_pad], agg_p[n_pad:],
                           ssum_p[:n_pad], ssum_p[n_pad:],
                           item_p[:n_pad], item_p[n_pad:])
        usr = _tc_add2(user_p[:n_pad], user_p[n_pad:])
        user_embs.append(usr)
        entity_embs.append(ent)

    user_out = _tc_mean3(*user_embs)
    entity_out = _tc_mean3(*entity_embs)
    return user_out, entity_out


# speedup vs baseline: 4.5393x; 1.0045x over previous
import functools
import math

import jax
import jax.numpy as jnp
from jax import lax
from jax.experimental import pallas as pl
from jax.experimental.pallas import tpu as pltpu
from jax.experimental.pallas import tpu_sc as plsc

N_ENT = 10000
N_USR = 10000
DIMS = 128
N_REL = 16
N_HEADS = 2
D_K = DIMS // N_HEADS
LAYERS = 2

NC = 2
NS = 16
NW = NC * NS
CH = 128
SB = 16
CB = 2048

_MESH = dict(core_axis_name="c", subcore_axis_name="s", num_cores=NC,
             num_subcores=NS)


@functools.partial(jax.jit, static_argnames=("k_per_w", "d"))
def _sc_gather(table, idx2d, *, k_per_w, d):
    b = NW * k_per_w * CH
    v = table.shape[0]
    rv = (v // NS) // 8 * 8
    rv_last = v - rv * (NS - 1)
    mesh = plsc.VectorSubcoreMesh(**_MESH)

    @functools.partial(
        pl.kernel,
        out_type=jax.ShapeDtypeStruct((b, d), jnp.float32),
        mesh=mesh,
        scratch_types=[
            pltpu.VMEM((k_per_w, CH), jnp.int32),
            pltpu.VMEM((CH, d), jnp.float32),
            pltpu.VMEM((CH, d), jnp.float32),
            pltpu.VMEM_SHARED((v, d), jnp.float32),
            pltpu.SemaphoreType.DMA,
            pltpu.SemaphoreType.DMA,
            pltpu.SemaphoreType.DMA,
            pltpu.SemaphoreType.DMA,
        ],
        compiler_params=pltpu.CompilerParams(needs_layout_passes=False),
    )
    def k(table_hbm, idx_hbm, out_hbm, idx_v, rows0, rows1, tab_sh, sg0,
          sg1, ss0, ss1):
        s = lax.axis_index("s")
        wid = s * NC + lax.axis_index("c")

        @pl.when(s < NS - 1)
        def _():
            pltpu.sync_copy(table_hbm.at[pl.ds(s * rv, rv)],
                            tab_sh.at[pl.ds(s * rv, rv)])

        @pl.when(s == NS - 1)
        def _():
            pltpu.sync_copy(table_hbm.at[pl.ds((NS - 1) * rv, rv_last)],
                            tab_sh.at[pl.ds((NS - 1) * rv, rv_last)])

        pltpu.sync_copy(idx_hbm.at[pl.ds(wid * k_per_w, k_per_w)], idx_v)
        plsc.subcore_barrier()
        bufs = ((rows0, sg0, ss0), (rows1, sg1, ss1))

        def gather_start(j, buf, sg):
            pltpu.async_copy(tab_sh.at[idx_v.at[j]], buf, sg)

        def store_start(j, buf, ss):
            pltpu.async_copy(
                buf, out_hbm.at[pl.ds((wid * k_per_w + j) * CH, CH)], ss)

        gather_start(0, rows0, sg0)
        gather_start(1, rows1, sg1)

        def step(i, carry):
            jj = 2 * i
            for b in range(2):
                buf, sg, ss = bufs[b]
                pltpu.make_async_copy(table_hbm.at[idx_v.at[0]], buf,
                                      sg).wait()
                store_start(jj + b, buf, ss)
            for b in range(2):
                buf, sg, ss = bufs[b]

                @pl.when(jj + b + 2 < k_per_w)
                def _():
                    pltpu.make_async_copy(
                        buf, out_hbm.at[pl.ds(0, CH)], ss).wait()
                    gather_start(jj + b + 2, buf, sg)

            return carry

        lax.fori_loop(0, k_per_w // 2, step, 0)
        for b in range(2):
            buf, sg, ss = bufs[b]
            pltpu.make_async_copy(buf, out_hbm.at[pl.ds(0, CH)], ss).wait()

    return k(table, idx2d)


@functools.partial(jax.jit, static_argnames=("k_per_w", "n_pad"))
def _sc_scatter_add(values, idx2d, zeros, *, k_per_w, n_pad):
    mesh = plsc.VectorSubcoreMesh(**_MESH)
    rpt = n_pad // NS

    @functools.partial(
        pl.kernel,
        out_type=jax.ShapeDtypeStruct((NC * n_pad, DIMS), jnp.float32),
        mesh=mesh,
        scratch_types=[
            pltpu.VMEM((k_per_w, CH), jnp.int32),
            pltpu.VMEM((CH, DIMS), jnp.float32),
            pltpu.VMEM((CH, DIMS), jnp.float32),
            pltpu.VMEM_SHARED((n_pad, DIMS), jnp.float32),
            pltpu.SemaphoreType.DMA,
            pltpu.SemaphoreType.DMA,
            pltpu.SemaphoreType.DMA,
            pltpu.SemaphoreType.DMA,
        ],
        compiler_params=pltpu.CompilerParams(needs_layout_passes=False),
    )
    def k(val_hbm, idx_hbm, zero_hbm, out_hbm, idx_v, val0, val1, acc_sh,
          sl0, sl1, sc0, sc1):
        c = lax.axis_index("c")
        s = lax.axis_index("s")
        wid = s * NC + c
        pltpu.sync_copy(zero_hbm.at[pl.ds(s * rpt, rpt)],
                        acc_sh.at[pl.ds(s * rpt, rpt)])
        pltpu.sync_copy(idx_hbm.at[pl.ds(wid * k_per_w, k_per_w)], idx_v)
        plsc.subcore_barrier()
        bufs = ((val0, sl0, sc0), (val1, sl1, sc1))

        def load_start(j, buf, sl):
            pltpu.async_copy(
                val_hbm.at[pl.ds((wid * k_per_w + j) * CH, CH)], buf, sl)

        load_start(0, val0, sl0)
        load_start(1, val1, sl1)

        def step(i, carry):
            jj = 2 * i
            for b in range(2):
                buf, sl, sc = bufs[b]
                pltpu.make_async_copy(val_hbm.at[pl.ds(0, CH)], buf,
                                      sl).wait()
                pltpu.async_copy(buf, acc_sh.at[idx_v.at[jj + b]], sc,
                                 add=True)
            for b in range(2):
                buf, sl, sc = bufs[b]

                @pl.when(jj + b + 2 < k_per_w)
                def _():
                    pltpu.make_async_copy(buf, acc_sh.at[idx_v.at[0]],
                                          sc).wait()
                    load_start(jj + b + 2, buf, sl)

            return carry

        lax.fori_loop(0, k_per_w // 2, step, 0)
        for b in range(2):
            buf, sl, sc = bufs[b]
            pltpu.make_async_copy(buf, acc_sh.at[idx_v.at[0]], sc).wait()
        plsc.subcore_barrier()
        pltpu.sync_copy(acc_sh.at[pl.ds(s * rpt, rpt)],
                        out_hbm.at[pl.ds(c * n_pad + s * rpt, rpt)])

    return k(values, idx2d, zeros)


@functools.partial(jax.jit, static_argnames=("k_per_w", "n_pad"))
def _sc_ssum_scatter(ex2, idx2d, zeros, *, k_per_w, n_pad):
    mesh = plsc.VectorSubcoreMesh(**_MESH)
    rpt = n_pad // NS
    nex = k_per_w * CH * 2

    @functools.partial(
        pl.kernel,
        out_type=jax.ShapeDtypeStruct((NC * n_pad, DIMS), jnp.float32),
        mesh=mesh,
        scratch_types=[
            pltpu.VMEM((SB, CH), jnp.int32),
            pltpu.VMEM((SB, 2 * CH), jnp.float32),
            pltpu.VMEM((CH, DIMS), jnp.float32),
            pltpu.VMEM((CH, DIMS), jnp.float32),
            pltpu.VMEM_SHARED((n_pad, DIMS), jnp.float32),
            pltpu.SemaphoreType.DMA,
            pltpu.SemaphoreType.DMA,
        ],
        compiler_params=pltpu.CompilerParams(needs_layout_passes=False),
    )
    def k(ex_hbm, idx_hbm, zero_hbm, out_hbm, idx_v, ex_v, val0, val1,
          acc_sh, sc0, sc1):
        c = lax.axis_index("c")
        s = lax.axis_index("s")
        wid = s * NC + c
        pltpu.sync_copy(zero_hbm.at[pl.ds(s * rpt, rpt)],
                        acc_sh.at[pl.ds(s * rpt, rpt)])
        bufs = ((val0, sc0), (val1, sc1))

        def zrow(i, carry):
            for t in range(DIMS // 16):
                z = jnp.zeros((16,), jnp.float32)
                val0[i, pl.ds(t * 16, 16)] = z
                val1[i, pl.ds(t * 16, 16)] = z
            return carry

        lax.fori_loop(0, CH, zrow, 0)
        plsc.subcore_barrier()
        lane = lax.iota(jnp.int32, 16)
        c0 = (lane == 0).astype(jnp.float32)
        c1 = (lane == 1).astype(jnp.float32)

        def fill(j, buf):
            def fill_grp(g, cc):
                for half in range(2):
                    p = ex_v[j, pl.ds(32 * g + 16 * half, 16)]
                    for l in range(8):
                        e = 16 * g + 8 * half + l
                        v0 = jnp.sum(p * (lane == 2 * l).astype(jnp.float32))
                        v1 = jnp.sum(p * (lane == 2 * l + 1).astype(
                            jnp.float32))
                        buf[e, pl.ds(0, 16)] = (jnp.full((16,), v0) * c0 +
                                                jnp.full((16,), v1) * c1)
                return cc

            lax.fori_loop(0, CH // 16, fill_grp, 0)

        def seg(h, carry):
            @pl.when(h > 0)
            def _():
                for b in range(2):
                    buf, sc = bufs[b]
                    pltpu.make_async_copy(buf, acc_sh.at[idx_v.at[0]],
                                          sc).wait()

            base = wid * k_per_w + h * SB
            pltpu.sync_copy(idx_hbm.at[pl.ds(base, SB)], idx_v)
            pltpu.sync_copy(ex_hbm.at[pl.ds(base, SB)], ex_v)

            def step(i, cc):
                jj = 2 * i
                for b in range(2):
                    buf, sc = bufs[b]
                    j = jj + b

                    @pl.when(j >= 2)
                    def _():
                        pltpu.make_async_copy(buf, acc_sh.at[idx_v.at[0]],
                                              sc).wait()

                    fill(j, buf)
                    pltpu.async_copy(buf, acc_sh.at[idx_v.at[j]], sc,
                                     add=True)
                return cc

            lax.fori_loop(0, SB // 2, step, 0)
            return carry

        lax.fori_loop(0, k_per_w // SB, seg, 0)
        for b in range(2):
            buf, sc = bufs[b]
            pltpu.make_async_copy(buf, acc_sh.at[idx_v.at[0]], sc).wait()
        plsc.subcore_barrier()
        pltpu.sync_copy(acc_sh.at[pl.ds(s * rpt, rpt)],
                        out_hbm.at[pl.ds(c * n_pad + s * rpt, rpt)])

    return k(ex2.reshape(NW * k_per_w, 2 * CH), idx2d, zeros)


def _proj_body(ent_ref, wq_ref, out_ref):
    out_ref[...] = jnp.dot(ent_ref[...], wq_ref[...],
                           preferred_element_type=jnp.float32)


@jax.jit
def _tc_project(ent, wq):
    return pl.pallas_call(
        _proj_body,
        out_shape=jax.ShapeDtypeStruct(ent.shape, jnp.float32),
    )(ent, wq)


def _score_body(e_total, qh_ref, tail_ref, ridx_ref, rel_ref, wq_ref,
                ex_ref, uval_ref):
    i = pl.program_id(0)
    q = qh_ref[0]
    tail = tail_ref[0]
    ridx = ridx_ref[0, 0]
    onehot = (ridx[:, None] ==
              lax.broadcasted_iota(jnp.int32, (CB, N_REL), 1)).astype(
                  jnp.float32)
    relrow = jnp.dot(onehot, rel_ref[...], preferred_element_type=jnp.float32)
    tw = jnp.dot(tail, wq_ref[...], preferred_element_type=jnp.float32)
    prod = q * tw * relrow
    s0 = jnp.sum(prod[:, :D_K], axis=1) * (1.0 / math.sqrt(D_K))
    s1 = jnp.sum(prod[:, D_K:], axis=1) * (1.0 / math.sqrt(D_K))
    ex = jnp.exp(jnp.stack([s0, s1], axis=1))
    eid2 = i * CB + lax.broadcasted_iota(jnp.int32, (CB, 2), 0)
    ex = jnp.where(eid2 < e_total, ex, 0.0)
    ex_ref[0] = ex
    aexp = jnp.concatenate(
        [jnp.repeat(ex[:, 0:1], D_K, axis=1),
         jnp.repeat(ex[:, 1:2], D_K, axis=1)], axis=1)
    uval_ref[0] = tail * relrow * aexp


@functools.partial(jax.jit, static_argnames=("e_total",))
def _tc_scores(qh, tail, ridx3, rel, wq, *, e_total):
    gb = qh.shape[0]
    return pl.pallas_call(
        functools.partial(_score_body, e_total),
        grid=(gb,),
        in_specs=[
            pl.BlockSpec((1, CB, DIMS), lambda i: (i, 0, 0)),
            pl.BlockSpec((1, CB, DIMS), lambda i: (i, 0, 0)),
            pl.BlockSpec((1, 1, CB), lambda i: (i, 0, 0)),
            pl.BlockSpec((N_REL, DIMS), lambda i: (0, 0)),
            pl.BlockSpec((DIMS, DIMS), lambda i: (0, 0)),
        ],
        out_specs=[
            pl.BlockSpec((1, CB, 2), lambda i: (i, 0, 0)),
            pl.BlockSpec((1, CB, DIMS), lambda i: (i, 0, 0)),
        ],
        out_shape=[
            jax.ShapeDtypeStruct((gb, CB, 2), jnp.float32),
            jax.ShapeDtypeStruct((gb, CB, DIMS), jnp.float32),
        ],
    )(qh, tail, ridx3, rel, wq)


def _scale_body(rows_ref, w_ref, out_ref):
    out_ref[0] = rows_ref[0] * w_ref[0, 0][:, None]


@jax.jit
def _tc_scale_rows(rows3, w3):
    gb = rows3.shape[0]
    return pl.pallas_call(
        _scale_body,
        grid=(gb,),
        in_specs=[
            pl.BlockSpec((1, CB, DIMS), lambda i: (i, 0, 0)),
            pl.BlockSpec((1, 1, CB), lambda i: (i, 0, 0)),
        ],
        out_specs=pl.BlockSpec((1, CB, DIMS), lambda i: (i, 0, 0)),
        out_shape=jax.ShapeDtypeStruct((gb, CB, DIMS), jnp.float32),
    )(rows3, w3)


def _finalize_body(agg0_ref, agg1_ref, ss0_ref, ss1_ref, it0_ref, it1_ref,
                   ent_ref):
    agg = agg0_ref[pl.ds(0, N_ENT)] + agg1_ref[pl.ds(0, N_ENT)]
    ss = ss0_ref[pl.ds(0, N_ENT)] + ss1_ref[pl.ds(0, N_ENT)]
    den0 = ss[:, 0:1] + 1e-16
    den1 = ss[:, 1:2] + 1e-16
    agg = jnp.concatenate(
        [agg[:, :D_K] / den0, agg[:, D_K:] / den1], axis=1)
    n = jnp.sqrt(jnp.sum(agg * agg, axis=1, keepdims=True))
    agg = agg / jnp.maximum(n, 1e-12)
    ent_ref[...] = agg + it0_ref[pl.ds(0, N_ENT)] + it1_ref[pl.ds(0, N_ENT)]


@jax.jit
def _tc_finalize(agg0, agg1, ss0, ss1, it0, it1):
    return pl.pallas_call(
        _finalize_body,
        out_shape=jax.ShapeDtypeStruct((N_ENT, DIMS), jnp.float32),
    )(agg0, agg1, ss0, ss1, it0, it1)


def _add2_body(a_ref, b_ref, o_ref):
    o_ref[...] = a_ref[pl.ds(0, N_USR)] + b_ref[pl.ds(0, N_USR)]


@jax.jit
def _tc_add2(a, b):
    return pl.pallas_call(
        _add2_body,
        out_shape=jax.ShapeDtypeStruct((N_USR, DIMS), jnp.float32),
    )(a, b)


def _mean3_body(a_ref, b_ref, c_ref, o_ref):
    o_ref[...] = (a_ref[...] + b_ref[...] + c_ref[...]) * (1.0 / 3.0)


@jax.jit
def _tc_mean3(a, b, c):
    return pl.pallas_call(
        _mean3_body,
        out_shape=jax.ShapeDtypeStruct(a.shape, jnp.float32),
    )(a, b, c)


def _pad_edges(x, e_pad, fill=0):
    e = x.shape[0]
    if e == e_pad:
        return x
    return jnp.concatenate(
        [x, jnp.full((e_pad - e,) + x.shape[1:], fill, x.dtype)])


def kernel(layers_num, user_emb, entity_emb, inter_edge, inter_edge_w,
           edge_index, edge_type, relation_emb, W_Q):
    e_kg = edge_index.shape[1]
    e_int = inter_edge.shape[1]
    k_per_w = ((-(-e_kg // (NW * CH)) + 7) // 8) * 8
    e_pad = NW * k_per_w * CH
    ki_per_w = ((-(-e_int // (NW * CH)) + 7) // 8) * 8
    ei_pad = NW * ki_per_w * CH
    gb = e_pad // CB
    n_pad = ((max(N_ENT, N_USR) + 127) // 128) * 128

    head = _pad_edges(edge_index[0].astype(jnp.int32), e_pad)
    tail_i = _pad_edges(edge_index[1].astype(jnp.int32), e_pad)
    ridx = _pad_edges(((edge_type.astype(jnp.int32) - 1) % N_REL), e_pad)
    iu = _pad_edges(inter_edge[0].astype(jnp.int32), ei_pad)
    ii = _pad_edges(inter_edge[1].astype(jnp.int32), ei_pad)
    iw = _pad_edges(inter_edge_w.astype(jnp.float32), ei_pad)

    head2d = head.reshape(NW * k_per_w, CH)
    tail2d = tail_i.reshape(NW * k_per_w, CH)
    ridx3 = ridx.reshape(gb, 1, CB)
    gbi = ei_pad // CB
    iu2d = iu.reshape(NW * ki_per_w, CH)
    ii2d = ii.reshape(NW * ki_per_w, CH)
    iw3 = iw.reshape(gbi, 1, CB)

    zeros128 = jnp.zeros((n_pad, DIMS), jnp.float32)

    usr = user_emb.astype(jnp.float32)
    ent = entity_emb.astype(jnp.float32)
    rel = relation_emb.astype(jnp.float32)
    wq = W_Q.astype(jnp.float32)

    user_embs = [usr]
    entity_embs = [ent]
    for _ in range(LAYERS):
        eq = _tc_project(ent, wq)
        qh = _sc_gather(eq, head2d, k_per_w=k_per_w, d=DIMS)
        tail_rows = _sc_gather(ent, tail2d, k_per_w=k_per_w, d=DIMS)
        qh3 = qh.reshape(gb, CB, DIMS)
        tail3 = tail_rows.reshape(gb, CB, DIMS)
        ex, uval = _tc_scores(qh3, tail3, ridx3, rel, wq, e_total=e_kg)
        agg_p = _sc_scatter_add(uval.reshape(e_pad, DIMS), head2d, zeros128,
                                k_per_w=k_per_w, n_pad=n_pad)
        ssum_p = _sc_ssum_scatter(ex.reshape(e_pad, 2), head2d, zeros128,
                                  k_per_w=k_per_w, n_pad=n_pad)
        item_rows = _sc_gather(ent, ii2d, k_per_w=ki_per_w, d=DIMS)
        user_rows = _sc_gather(usr, iu2d, k_per_w=ki_per_w, d=DIMS)
        item_sc = _tc_scale_rows(item_rows.reshape(gbi, CB, DIMS), iw3)
        user_sc = _tc_scale_rows(user_rows.reshape(gbi, CB, DIMS), iw3)
        user_p = _sc_scatter_add(item_sc.reshape(ei_pad, DIMS), iu2d,
                                 zeros128, k_per_w=ki_per_w, n_pad=n_pad)
        item_p = _sc_scatter_add(user_sc.reshape(ei_pad, DIMS), ii2d,
                                 zeros128, k_per_w=ki_per_w, n_pad=n_pad)
        ent = _tc_finalize(agg_p[:n_pad], agg_p[n_pad:],
                           ssum_p[:n_pad], ssum_p[n_pad:],
                           item_p[:n_pad], item_p[n_pad:])
        usr = _tc_add2(user_p[:n_pad], user_p[n_pad:])
        user_embs.append(usr)
        entity_embs.append(ent)

    user_out = _tc_mean3(*user_embs)
    entity_out = _tc_mean3(*entity_embs)
    return user_out, entity_out
